# Initial kernel scaffold; baseline (speedup 1.0000x reference)
import jax
import jax.numpy as jnp
from jax.experimental import pallas as pl


def kernel(x, edge_index, edge_attr, atom_table, bond_table, Wm, Wn, We, pool_p):
    raise NotImplementedError("write your pallas kernel here")


# speedup vs baseline: 1.0341x; 1.0341x over previous
import functools

import jax
import jax.numpy as jnp
from jax.experimental import pallas as pl
from jax.experimental.pallas import tpu as pltpu

N = 10000
E = 160000
D = 128
L = 3

_BN = 2000
_BE = 8000


def _mm_body(x_ref, w_ref, o_ref):
    o_ref[...] = jnp.dot(x_ref[...], w_ref[...],
                         preferred_element_type=jnp.float32)


def _node_mm(x, w):
    return pl.pallas_call(
        _mm_body,
        grid=(N // _BN,),
        in_specs=[
            pl.BlockSpec((_BN, D), lambda i: (i, 0)),
            pl.BlockSpec((D, D), lambda i: (0, 0)),
        ],
        out_specs=pl.BlockSpec((_BN, D), lambda i: (i, 0)),
        out_shape=jax.ShapeDtypeStruct((N, D), jnp.float32),
    )(x, w)


def _hnew_body(h_ref, agg_ref, wn_ref, we_ref, hn_ref, hwe_ref):
    hn = jnp.maximum(
        jnp.dot(h_ref[...] + agg_ref[...], wn_ref[...],
                preferred_element_type=jnp.float32), 0.0)
    hn_ref[...] = hn
    hwe_ref[...] = jnp.dot(hn, we_ref[...],
                           preferred_element_type=jnp.float32)


def _hnew_fused(h, agg, wn, we):
    return pl.pallas_call(
        _hnew_body,
        grid=(N // _BN,),
        in_specs=[
            pl.BlockSpec((_BN, D), lambda i: (i, 0)),
            pl.BlockSpec((_BN, D), lambda i: (i, 0)),
            pl.BlockSpec((D, D), lambda i: (0, 0)),
            pl.BlockSpec((D, D), lambda i: (0, 0)),
        ],
        out_specs=[
            pl.BlockSpec((_BN, D), lambda i: (i, 0)),
            pl.BlockSpec((_BN, D), lambda i: (i, 0)),
        ],
        out_shape=[
            jax.ShapeDtypeStruct((N, D), jnp.float32),
            jax.ShapeDtypeStruct((N, D), jnp.float32),
        ],
    )(h, agg, wn, we)


def _edge_body(e_ref, g_ref, w_ref, o_ref):
    o_ref[...] = jnp.maximum(
        jnp.dot(e_ref[...], w_ref[...],
                preferred_element_type=jnp.float32) + g_ref[...], 0.0)


def _edge_mm(e, g23, we):
    return pl.pallas_call(
        _edge_body,
        grid=(E // _BE,),
        in_specs=[
            pl.BlockSpec((_BE, D), lambda i: (i, 0)),
            pl.BlockSpec((_BE, D), lambda i: (i, 0)),
            pl.BlockSpec((D, D), lambda i: (0, 0)),
        ],
        out_specs=pl.BlockSpec((_BE, D), lambda i: (i, 0)),
        out_shape=jax.ShapeDtypeStruct((E, D), jnp.float32),
    )(e, g23, we)


def _relu_add_body(a_ref, b_ref, o_ref):
    o_ref[...] = jnp.maximum(a_ref[...] + b_ref[...], 0.0)


def _relu_add(a, b):
    return pl.pallas_call(
        _relu_add_body,
        grid=(E // _BE,),
        in_specs=[
            pl.BlockSpec((_BE, D), lambda i: (i, 0)),
            pl.BlockSpec((_BE, D), lambda i: (i, 0)),
        ],
        out_specs=pl.BlockSpec((_BE, D), lambda i: (i, 0)),
        out_shape=jax.ShapeDtypeStruct((E, D), jnp.float32),
    )(a, b)


def _mp(i, h, e, src, dst, Wm, Wn, We):
    hWm = _node_mm(h, Wm[i])
    m = _relu_add(jnp.take(hWm, src, axis=0), e)
    agg = jax.ops.segment_sum(m, dst, num_segments=N)
    h_new, hWe = _hnew_fused(h, agg, Wn[i], We[i])
    g23 = jnp.take(hWe, src, axis=0) + jnp.take(hWe, dst, axis=0)
    e_new = _edge_mm(e, g23, We[i])
    return h_new, e_new


def kernel(x, edge_index, edge_attr, atom_table, bond_table, Wm, Wn, We, pool_p):
    src = edge_index[0]
    dst = edge_index[1]
    h = jnp.take(atom_table, x, axis=0)
    e = jnp.take(bond_table, edge_attr, axis=0)
    h, e = _mp(0, h, e, src, dst, Wm, Wn, We)
    xs = []
    emx = []
    eme = []
    pool_features = []
    for i in range(L):
        xs.append(h)
        p = pool_p[i]
        score = (h @ p) / jnp.linalg.norm(p)
        h = h * jnp.tanh(score)[:, None]
        h, e = _mp(1 + i, h, e, src, dst, Wm, Wn, We)
        pool_features.append(h)
        gx, ge = h, e
        w = 1.0
        base = 1 + L + i * (i + 1) // 2
        for j in range(i, -1, -1):
            gx = gx + xs[j] / w
            gx, ge = _mp(base + j, gx, ge, src, dst, Wm, Wn, We)
            xs[j] = xs[j] + gx
            w += 1.0
        emx.append(gx)
        eme.append(ge)
    return (jnp.stack(emx), jnp.stack(eme), jnp.stack(pool_features))


# speedup vs baseline: 2.4461x; 2.3655x over previous
import functools

import jax
import jax.numpy as jnp
from jax import lax
from jax.experimental import pallas as pl
from jax.experimental.pallas import tpu as pltpu
from jax.experimental.pallas import tpu_sc as plsc

N = 10000
E = 160000
D = 128
L = 3

_BN = 2000
_BE = 8000

_NC, _NS = 2, 16
_NW = _NC * _NS
_CA = 40
_CB = 128
_RPT = 624
_REM0, _REMN = _NS * _RPT, N - _NS * _RPT

_sc_mesh = plsc.VectorSubcoreMesh(core_axis_name="c", subcore_axis_name="s")


def _mm_body(x_ref, w_ref, o_ref):
    o_ref[...] = jnp.dot(x_ref[...], w_ref[...],
                         preferred_element_type=jnp.float32)


def _node_mm(x, w):
    return pl.pallas_call(
        _mm_body,
        grid=(N // _BN,),
        in_specs=[
            pl.BlockSpec((_BN, D), lambda i: (i, 0)),
            pl.BlockSpec((D, D), lambda i: (0, 0)),
        ],
        out_specs=pl.BlockSpec((_BN, D), lambda i: (i, 0)),
        out_shape=jax.ShapeDtypeStruct((N, D), jnp.float32),
    )(x, w)


def _edge_mm(x, w):
    return pl.pallas_call(
        _mm_body,
        grid=(E // _BE,),
        in_specs=[
            pl.BlockSpec((_BE, D), lambda i: (i, 0)),
            pl.BlockSpec((D, D), lambda i: (0, 0)),
        ],
        out_specs=pl.BlockSpec((_BE, D), lambda i: (i, 0)),
        out_shape=jax.ShapeDtypeStruct((E, D), jnp.float32),
    )(x, w)


def _small_mm(x, w):
    m, _ = x.shape
    return pl.pallas_call(
        _mm_body,
        in_specs=[pl.BlockSpec(x.shape, lambda: (0, 0)),
                  pl.BlockSpec(w.shape, lambda: (0, 0))],
        out_specs=pl.BlockSpec((m, w.shape[1]), lambda: (0, 0)),
        out_shape=jax.ShapeDtypeStruct((m, w.shape[1]), jnp.float32),
    )(x, w)


def _hnew_body(h_ref, a0_ref, a1_ref, wn_ref, we_ref, hn_ref, hwe_ref):
    hn = jnp.maximum(
        jnp.dot(h_ref[...] + (a0_ref[...] + a1_ref[...]), wn_ref[...],
                preferred_element_type=jnp.float32), 0.0)
    hn_ref[...] = hn
    hwe_ref[...] = jnp.dot(hn, we_ref[...],
                           preferred_element_type=jnp.float32)


def _hnew_fused(h, aggp, wn, we):
    nb = N // _BN
    return pl.pallas_call(
        _hnew_body,
        grid=(nb,),
        in_specs=[
            pl.BlockSpec((_BN, D), lambda i: (i, 0)),
            pl.BlockSpec((_BN, D), lambda i: (i, 0)),
            pl.BlockSpec((_BN, D), lambda i: (i + nb, 0)),
            pl.BlockSpec((D, D), lambda i: (0, 0)),
            pl.BlockSpec((D, D), lambda i: (0, 0)),
        ],
        out_specs=[
            pl.BlockSpec((_BN, D), lambda i: (i, 0)),
            pl.BlockSpec((_BN, D), lambda i: (i, 0)),
        ],
        out_shape=[
            jax.ShapeDtypeStruct((N, D), jnp.float32),
            jax.ShapeDtypeStruct((N, D), jnp.float32),
        ],
    )(h, aggp, aggp, wn, we)


def _emb_body(ids_ref, tab_ref, o_ref):
    oh = (ids_ref[...] == lax.broadcasted_iota(jnp.int32, (1, 128), 1))
    o_ref[...] = jnp.dot(oh.astype(jnp.float32), tab_ref[...],
                         preferred_element_type=jnp.float32)


def _emb_mm(ids2d, tab_pad):
    return pl.pallas_call(
        _emb_body,
        grid=(N // _BN,),
        in_specs=[
            pl.BlockSpec((_BN, 1), lambda i: (i, 0)),
            pl.BlockSpec((128, D), lambda i: (0, 0)),
        ],
        out_specs=pl.BlockSpec((_BN, D), lambda i: (i, 0)),
        out_shape=jax.ShapeDtypeStruct((N, D), jnp.float32),
    )(ids2d, tab_pad)


def _relu_sum_rows(nrows, dst_ref, a_ref, b_ref=None):
    def body(r, carry):
        for u in range(D // 16):
            sl = pl.ds(u * 16, 16)
            v = dst_ref[r, sl] + a_ref[r, sl]
            if b_ref is not None:
                v = v + b_ref[r, sl]
            dst_ref[r, sl] = jnp.maximum(v, 0.0)
        return carry
    lax.fori_loop(0, nrows, body, 0)


def _make_sca(e_from_table: bool):
    C = _CA
    kfull = E // C // _NW
    assert kfull % 2 == 1 and kfull * _NW * C == E

    scratch = [
        pltpu.VMEM((C,), jnp.int32),
        pltpu.VMEM((C,), jnp.int32),
        pltpu.VMEM((C,), jnp.int32),
        pltpu.VMEM((C,), jnp.int32),
        pltpu.VMEM((C, D), jnp.float32),
        pltpu.VMEM((C, D), jnp.float32),
        pltpu.VMEM((C, D), jnp.float32),
        pltpu.VMEM((C, D), jnp.float32),
        pltpu.VMEM_SHARED((N, D), jnp.float32),
        pltpu.SemaphoreType.DMA,
        pltpu.SemaphoreType.DMA,
        pltpu.SemaphoreType.DMA,
        pltpu.SemaphoreType.DMA,
    ]
    if e_from_table:
        scratch = [pltpu.VMEM((C,), jnp.int32),
                   pltpu.VMEM((C,), jnp.int32)] + scratch

    def body(*refs):
        if e_from_table:
            hwm, etab, eattr, src, dst, out = refs[:6]
            ie = refs[6:8]
            rest = refs[8:]
        else:
            hwm, esrc, src, dst, out = refs[:5]
            rest = refs[5:]
        isb = rest[0:2]
        idb = rest[2:4]
        eb = rest[4:6]
        rb = rest[6:8]
        agg = rest[8]
        se = rest[9:11]
        sg = rest[11:13]

        cid = lax.axis_index("c")
        sid = lax.axis_index("s")
        wid = cid * _NS + sid

        def zbody(r, carry):
            for u in range(D // 16):
                rb[0][r, pl.ds(u * 16, 16)] = jnp.zeros((16,), jnp.float32)
            return carry
        lax.fori_loop(0, C, zbody, 0)
        nz, rem = _RPT // C, _RPT % C
        for j in range(nz):
            pltpu.sync_copy(rb[0].at[pl.ds(0, C)],
                            agg.at[pl.ds(sid * _RPT + j * C, C)])
        if rem:
            pltpu.sync_copy(rb[0].at[pl.ds(0, rem)],
                            agg.at[pl.ds(sid * _RPT + nz * C, rem)])

        @pl.when(sid == _NS - 1)
        def _():
            pltpu.sync_copy(rb[0].at[pl.ds(0, _REMN)],
                            agg.at[pl.ds(_REM0, _REMN)])
        plsc.subcore_barrier()

        def issue(k, t):
            base = pl.multiple_of((k * _NW + wid) * C, C)
            pltpu.sync_copy(src.at[pl.ds(base, C)], isb[t])
            pltpu.sync_copy(dst.at[pl.ds(base, C)], idb[t])
            if e_from_table:
                pltpu.sync_copy(eattr.at[pl.ds(base, C)], ie[t])
                pltpu.make_async_copy(etab.at[ie[t]], eb[t], se[t]).start()
            else:
                pltpu.make_async_copy(esrc.at[pl.ds(base, C), :],
                                      eb[t], se[t]).start()
            pltpu.make_async_copy(hwm.at[isb[t]], rb[t], sg[t]).start()

        def finish(t):
            if e_from_table:
                pltpu.make_async_copy(etab.at[ie[t]], eb[t], se[t]).wait()
            else:
                pltpu.make_async_copy(esrc.at[pl.ds(0, C), :],
                                      eb[t], se[t]).wait()
            pltpu.make_async_copy(hwm.at[isb[t]], rb[t], sg[t]).wait()
            _relu_sum_rows(C, eb[t], rb[t])
            pltpu.sync_copy(eb[t], agg.at[idb[t]], add=True)

        issue(0, 0)

        def pair(j, carry):
            k0 = 2 * j
            issue(k0 + 1, 1)
            finish(0)
            issue(k0 + 2, 0)
            finish(1)
            return carry
        lax.fori_loop(0, (kfull - 1) // 2, pair, 0)
        finish(0)

        plsc.subcore_barrier()
        for off, sz in [(0, 128), (128, 128), (256, 128), (384, 128),
                        (512, 112)]:
            r0 = sid * _RPT + off
            pltpu.sync_copy(agg.at[pl.ds(r0, sz)],
                            out.at[pl.ds(cid * N + r0, sz)])

        @pl.when(sid == _NS - 1)
        def _():
            pltpu.sync_copy(agg.at[pl.ds(_REM0, _REMN)],
                            out.at[pl.ds(cid * N + _REM0, _REMN)])

    return functools.partial(
        pl.kernel,
        out_type=jax.ShapeDtypeStruct((2 * N, D), jnp.float32),
        mesh=_sc_mesh,
        scratch_types=scratch,
    )(body)


def _make_scb(ew_from_table: bool):
    C = _CB
    kfull = E // C // _NW
    nextra = E // C - kfull * _NW
    scratch = [
        pltpu.VMEM((C,), jnp.int32),
        pltpu.VMEM((C,), jnp.int32),
        pltpu.VMEM((C,), jnp.int32),
        pltpu.VMEM((C,), jnp.int32),
        pltpu.VMEM((C, D), jnp.float32),
        pltpu.VMEM((C, D), jnp.float32),
        pltpu.VMEM((C, D), jnp.float32),
        pltpu.VMEM((C, D), jnp.float32),
        pltpu.VMEM((C, D), jnp.float32),
        pltpu.VMEM((C, D), jnp.float32),
        pltpu.SemaphoreType.DMA,
        pltpu.SemaphoreType.DMA,
        pltpu.SemaphoreType.DMA,
        pltpu.SemaphoreType.DMA,
        pltpu.SemaphoreType.DMA,
        pltpu.SemaphoreType.DMA,
    ]
    if ew_from_table:
        scratch = [pltpu.VMEM((C,), jnp.int32),
                   pltpu.VMEM((C,), jnp.int32)] + scratch

    def body(*refs):
        if ew_from_table:
            ewtab, eattr, hwe, src, dst, out = refs[:6]
            ie = refs[6:8]
            rest = refs[8:]
        else:
            ew, hwe, src, dst, out = refs[:5]
            rest = refs[5:]
        isb = rest[0:2]
        idb = rest[2:4]
        eb = rest[4:6]
        b1 = rest[6:8]
        b2 = rest[8:10]
        se = rest[10:12]
        s1 = rest[12:14]
        s2 = rest[14:16]

        cid = lax.axis_index("c")
        sid = lax.axis_index("s")
        wid = cid * _NS + sid

        def issue(k, t):
            base = pl.multiple_of((k * _NW + wid) * C, C)
            pltpu.sync_copy(src.at[pl.ds(base, C)], isb[t])
            pltpu.sync_copy(dst.at[pl.ds(base, C)], idb[t])
            if ew_from_table:
                pltpu.sync_copy(eattr.at[pl.ds(base, C)], ie[t])
                pltpu.make_async_copy(ewtab.at[ie[t]], eb[t], se[t]).start()
            else:
                pltpu.make_async_copy(ew.at[pl.ds(base, C), :],
                                      eb[t], se[t]).start()
            pltpu.make_async_copy(hwe.at[isb[t]], b1[t], s1[t]).start()
            pltpu.make_async_copy(hwe.at[idb[t]], b2[t], s2[t]).start()

        def finish(k, t):
            base = pl.multiple_of((k * _NW + wid) * C, C)
            if ew_from_table:
                pltpu.make_async_copy(ewtab.at[ie[t]], eb[t], se[t]).wait()
            else:
                pltpu.make_async_copy(ew.at[pl.ds(base, C), :],
                                      eb[t], se[t]).wait()
            pltpu.make_async_copy(hwe.at[isb[t]], b1[t], s1[t]).wait()
            pltpu.make_async_copy(hwe.at[idb[t]], b2[t], s2[t]).wait()
            _relu_sum_rows(C, eb[t], b1[t], b2[t])
            pltpu.sync_copy(eb[t], out.at[pl.ds(base, C), :])

        issue(0, 0)

        def pair(j, carry):
            k0 = 2 * j
            issue(k0 + 1, 1)
            finish(k0, 0)
            issue(k0 + 2, 0)
            finish(k0 + 1, 1)
            return carry
        lax.fori_loop(0, (kfull - 1) // 2, pair, 0)
        finish(kfull - 1, 0)

        @pl.when(wid < nextra)
        def _():
            issue(kfull, 1)
            finish(kfull, 1)

    return functools.partial(
        pl.kernel,
        out_type=jax.ShapeDtypeStruct((E, D), jnp.float32),
        mesh=_sc_mesh,
        scratch_types=scratch,
    )(body)


_sca = _make_sca(False)
_sca0 = _make_sca(True)
_scb = _make_scb(False)
_scb0 = _make_scb(True)


def _mp_block(i, h, e, eattr, btab, src, dst, Wm, Wn, We):
    hWm = _node_mm(h, Wm[i])
    if e is None:
        aggp = _sca0(hWm, btab, eattr, src, dst)
    else:
        ew = _edge_mm(e, We[i])
        aggp = _sca(hWm, e, src, dst)
    h_new, hWe = _hnew_fused(h, aggp, Wn[i], We[i])
    if e is None:
        bw = _small_mm(btab, We[i])
        e_new = _scb0(bw, eattr, hWe, src, dst)
    else:
        e_new = _scb(ew, hWe, src, dst)
    return h_new, e_new


def kernel(x, edge_index, edge_attr, atom_table, bond_table, Wm, Wn, We, pool_p):
    src = edge_index[0].astype(jnp.int32)
    dst = edge_index[1].astype(jnp.int32)
    eattr = edge_attr.astype(jnp.int32)
    atab_pad = jnp.zeros((128, D), jnp.float32).at[:119].set(atom_table)
    h = _emb_mm(x.astype(jnp.int32)[:, None], atab_pad)

    h, e = _mp_block(0, h, None, eattr, bond_table, src, dst, Wm, Wn, We)
    xs = []
    emx = []
    eme = []
    pool_features = []
    for i in range(L):
        xs.append(h)
        p = pool_p[i]
        score = (h @ p) / jnp.linalg.norm(p)
        h = h * jnp.tanh(score)[:, None]
        h, e = _mp_block(1 + i, h, e, eattr, bond_table, src, dst, Wm, Wn, We)
        pool_features.append(h)
        gx, ge = h, e
        w = 1.0
        base = 1 + L + i * (i + 1) // 2
        for j in range(i, -1, -1):
            gx = gx + xs[j] / w
            gx, ge = _mp_block(base + j, gx, ge, eattr, bond_table,
                               src, dst, Wm, Wn, We)
            xs[j] = xs[j] + gx
            w += 1.0
        emx.append(gx)
        eme.append(ge)
    return (jnp.stack(emx), jnp.stack(eme), jnp.stack(pool_features))


# speedup vs baseline: 4.3790x; 1.7902x over previous
import functools

import jax
import jax.numpy as jnp
from jax import lax
from jax.experimental import pallas as pl
from jax.experimental.pallas import tpu as pltpu
from jax.experimental.pallas import tpu_sc as plsc

N = 10000
E = 160000
D = 128
L = 3

_BN = 2000
_BE = 8000

_NC, _NS = 2, 16
_NW = _NC * _NS
_CA = 64
_CB = 128
_RPT = 624
_REM0, _REMN = _NS * _RPT, N - _NS * _RPT

_sc_mesh = plsc.VectorSubcoreMesh(core_axis_name="c", subcore_axis_name="s")


def _mm_body(x_ref, w_ref, o_ref):
    o_ref[...] = jnp.dot(x_ref[...], w_ref[...],
                         preferred_element_type=jnp.float32)


def _node_mm(x, w):
    return pl.pallas_call(
        _mm_body,
        grid=(N // _BN,),
        in_specs=[
            pl.BlockSpec((_BN, D), lambda i: (i, 0)),
            pl.BlockSpec((D, D), lambda i: (0, 0)),
        ],
        out_specs=pl.BlockSpec((_BN, D), lambda i: (i, 0)),
        out_shape=jax.ShapeDtypeStruct((N, D), jnp.float32),
    )(x, w)


def _edge_mm(x, w):
    return pl.pallas_call(
        _mm_body,
        grid=(E // _BE,),
        in_specs=[
            pl.BlockSpec((_BE, D), lambda i: (i, 0)),
            pl.BlockSpec((D, D), lambda i: (0, 0)),
        ],
        out_specs=pl.BlockSpec((_BE, D), lambda i: (i, 0)),
        out_shape=jax.ShapeDtypeStruct((E, D), jnp.float32),
    )(x, w)


def _small_mm(x, w):
    m, _ = x.shape
    return pl.pallas_call(
        _mm_body,
        in_specs=[pl.BlockSpec(x.shape, lambda: (0, 0)),
                  pl.BlockSpec(w.shape, lambda: (0, 0))],
        out_specs=pl.BlockSpec((m, w.shape[1]), lambda: (0, 0)),
        out_shape=jax.ShapeDtypeStruct((m, w.shape[1]), jnp.float32),
    )(x, w)


def _hnew_body(h_ref, a0_ref, a1_ref, wn_ref, we_ref, hn_ref, hwe_ref):
    hn = jnp.maximum(
        jnp.dot(h_ref[...] + (a0_ref[...] + a1_ref[...]), wn_ref[...],
                preferred_element_type=jnp.float32), 0.0)
    hn_ref[...] = hn
    hwe_ref[...] = jnp.dot(hn, we_ref[...],
                           preferred_element_type=jnp.float32)


def _hnew_fused(h, aggp, wn, we):
    nb = N // _BN
    return pl.pallas_call(
        _hnew_body,
        grid=(nb,),
        in_specs=[
            pl.BlockSpec((_BN, D), lambda i: (i, 0)),
            pl.BlockSpec((_BN, D), lambda i: (i, 0)),
            pl.BlockSpec((_BN, D), lambda i: (i + nb, 0)),
            pl.BlockSpec((D, D), lambda i: (0, 0)),
            pl.BlockSpec((D, D), lambda i: (0, 0)),
        ],
        out_specs=[
            pl.BlockSpec((_BN, D), lambda i: (i, 0)),
            pl.BlockSpec((_BN, D), lambda i: (i, 0)),
        ],
        out_shape=[
            jax.ShapeDtypeStruct((N, D), jnp.float32),
            jax.ShapeDtypeStruct((N, D), jnp.float32),
        ],
    )(h, aggp, aggp, wn, we)


def _emb_body(ids_ref, tab_ref, o_ref):
    oh = (ids_ref[...] == lax.broadcasted_iota(jnp.int32, (1, 128), 1))
    o_ref[...] = jnp.dot(oh.astype(jnp.float32), tab_ref[...],
                         preferred_element_type=jnp.float32)


def _e0_body(ids_ref, bt_ref, bw_ref, e0_ref, ew_ref):
    oh = (ids_ref[...] == lax.broadcasted_iota(jnp.int32, (1, 128), 1))
    ohf = oh.astype(jnp.float32)
    e0_ref[...] = jnp.dot(ohf, bt_ref[...], preferred_element_type=jnp.float32)
    ew_ref[...] = jnp.dot(ohf, bw_ref[...], preferred_element_type=jnp.float32)


def _e0_mm(eattr2d, btab_pad, bwe_pad):
    return pl.pallas_call(
        _e0_body,
        grid=(E // _BE,),
        in_specs=[
            pl.BlockSpec((_BE, 1), lambda i: (i, 0)),
            pl.BlockSpec((128, D), lambda i: (0, 0)),
            pl.BlockSpec((128, D), lambda i: (0, 0)),
        ],
        out_specs=[
            pl.BlockSpec((_BE, D), lambda i: (i, 0)),
            pl.BlockSpec((_BE, D), lambda i: (i, 0)),
        ],
        out_shape=[
            jax.ShapeDtypeStruct((E, D), jnp.float32),
            jax.ShapeDtypeStruct((E, D), jnp.float32),
        ],
    )(eattr2d, btab_pad, bwe_pad)


def _emb_mm(ids2d, tab_pad):
    return pl.pallas_call(
        _emb_body,
        grid=(N // _BN,),
        in_specs=[
            pl.BlockSpec((_BN, 1), lambda i: (i, 0)),
            pl.BlockSpec((128, D), lambda i: (0, 0)),
        ],
        out_specs=pl.BlockSpec((_BN, D), lambda i: (i, 0)),
        out_shape=jax.ShapeDtypeStruct((N, D), jnp.float32),
    )(ids2d, tab_pad)


def _relu_sum_rows(nrows, dst_ref, a_ref, b_ref=None):
    def body(r, carry):
        for u in range(D // 16):
            sl = pl.ds(u * 16, 16)
            v = dst_ref[r, sl] + a_ref[r, sl]
            if b_ref is not None:
                v = v + b_ref[r, sl]
            dst_ref[r, sl] = jnp.maximum(v, 0.0)
        return carry
    lax.fori_loop(0, nrows, body, 0)


def _make_sca():
    C = _CA
    kfull = E // C // _NW
    nextra = E // C - kfull * _NW
    assert kfull % 2 == 0 and E % C == 0

    scratch = [
        pltpu.VMEM((C,), jnp.int32),
        pltpu.VMEM((C,), jnp.int32),
        pltpu.VMEM((C,), jnp.int32),
        pltpu.VMEM((C,), jnp.int32),
        pltpu.VMEM((C,), jnp.int32),
        pltpu.VMEM((C,), jnp.int32),
        pltpu.VMEM((C, D), jnp.float32),
        pltpu.VMEM((C, D), jnp.float32),
        pltpu.VMEM((C, D), jnp.float32),
        pltpu.VMEM((C, D), jnp.float32),
        pltpu.VMEM((C, D), jnp.float32),
        pltpu.VMEM((C, D), jnp.float32),
        pltpu.VMEM_SHARED((N, D), jnp.float32),
        pltpu.SemaphoreType.DMA,
        pltpu.SemaphoreType.DMA,
        pltpu.SemaphoreType.DMA,
        pltpu.SemaphoreType.DMA,
        pltpu.SemaphoreType.DMA,
        pltpu.SemaphoreType.DMA,
    ]

    def body(hwm, esrc, src, dst, out, *rest):
        isb = rest[0:2]
        idb = rest[2:4]
        sdb = rest[4:6]
        eb = rest[6:8]
        rb = rest[8:10]
        sb = rest[10:12]
        agg = rest[12]
        se = rest[13:15]
        sg = rest[15:17]
        ss = rest[17:19]

        cid = lax.axis_index("c")
        sid = lax.axis_index("s")
        wid = cid * _NS + sid

        def zbody(r, carry):
            for u in range(D // 16):
                rb[0][r, pl.ds(u * 16, 16)] = jnp.zeros((16,), jnp.float32)
            return carry
        lax.fori_loop(0, C, zbody, 0)
        nz, rem = _RPT // C, _RPT % C
        for j in range(nz):
            pltpu.sync_copy(rb[0].at[pl.ds(0, C)],
                            agg.at[pl.ds(sid * _RPT + j * C, C)])
        if rem:
            pltpu.sync_copy(rb[0].at[pl.ds(0, rem)],
                            agg.at[pl.ds(sid * _RPT + nz * C, rem)])

        @pl.when(sid == _NS - 1)
        def _():
            pltpu.sync_copy(rb[0].at[pl.ds(0, _REMN)],
                            agg.at[pl.ds(_REM0, _REMN)])
        plsc.subcore_barrier()

        def issue(k, t):
            base = pl.multiple_of((k * _NW + wid) * C, C)
            pltpu.sync_copy(src.at[pl.ds(base, C)], isb[t])
            pltpu.sync_copy(dst.at[pl.ds(base, C)], idb[t])
            pltpu.make_async_copy(esrc.at[pl.ds(base, C), :],
                                  eb[t], se[t]).start()
            pltpu.make_async_copy(hwm.at[isb[t]], rb[t], sg[t]).start()

        def finish(t, guard_scatter=True):
            pltpu.make_async_copy(esrc.at[pl.ds(0, C), :],
                                  eb[t], se[t]).wait()
            pltpu.make_async_copy(hwm.at[isb[t]], rb[t], sg[t]).wait()
            if guard_scatter:
                pltpu.make_async_copy(sb[t], agg.at[sdb[t]], ss[t]).wait()
            for u in range(C // 16):
                sl = pl.ds(u * 16, 16)
                sdb[t][sl] = idb[t][sl]

            def cbody(r, carry):
                for u in range(D // 16):
                    sl = pl.ds(u * 16, 16)
                    sb[t][r, sl] = jnp.maximum(eb[t][r, sl] + rb[t][r, sl],
                                               0.0)
                return carry
            lax.fori_loop(0, C, cbody, 0)
            pltpu.async_copy(sb[t], agg.at[sdb[t]], ss[t], add=True)

        issue(0, 0)
        issue(1, 1)
        finish(0, guard_scatter=False)
        issue(2, 0)
        finish(1, guard_scatter=False)

        def pair(j, carry):
            k0 = 2 * j
            issue(k0 + 3, 1)
            finish(0)
            issue(k0 + 4, 0)
            finish(1)
            return carry
        lax.fori_loop(0, kfull // 2 - 2, pair, 0)
        issue(kfull - 1, 1)
        finish(0)
        if nextra:
            @pl.when(wid < nextra)
            def _():
                issue(kfull, 0)
            finish(1)

            @pl.when(wid < nextra)
            def _():
                finish(0)
        else:
            finish(1)

        pltpu.make_async_copy(sb[0], agg.at[sdb[0]], ss[0]).wait()
        pltpu.make_async_copy(sb[1], agg.at[sdb[1]], ss[1]).wait()

        plsc.subcore_barrier()
        for off, sz in [(0, 128), (128, 128), (256, 128), (384, 128),
                        (512, 112)]:
            r0 = sid * _RPT + off
            pltpu.sync_copy(agg.at[pl.ds(r0, sz)],
                            out.at[pl.ds(cid * N + r0, sz)])

        @pl.when(sid == _NS - 1)
        def _():
            pltpu.sync_copy(agg.at[pl.ds(_REM0, _REMN)],
                            out.at[pl.ds(cid * N + _REM0, _REMN)])

    return functools.partial(
        pl.kernel,
        out_type=jax.ShapeDtypeStruct((2 * N, D), jnp.float32),
        mesh=_sc_mesh,
        scratch_types=scratch,
    )(body)


def _make_scb(ew_from_table: bool):
    C = _CB
    kfull = E // C // _NW
    nextra = E // C - kfull * _NW
    scratch = [
        pltpu.VMEM((C,), jnp.int32),
        pltpu.VMEM((C,), jnp.int32),
        pltpu.VMEM((C,), jnp.int32),
        pltpu.VMEM((C,), jnp.int32),
        pltpu.VMEM((C, D), jnp.float32),
        pltpu.VMEM((C, D), jnp.float32),
        pltpu.VMEM((C, D), jnp.float32),
        pltpu.VMEM((C, D), jnp.float32),
        pltpu.VMEM((C, D), jnp.float32),
        pltpu.VMEM((C, D), jnp.float32),
        pltpu.SemaphoreType.DMA,
        pltpu.SemaphoreType.DMA,
        pltpu.SemaphoreType.DMA,
        pltpu.SemaphoreType.DMA,
        pltpu.SemaphoreType.DMA,
        pltpu.SemaphoreType.DMA,
    ]
    if ew_from_table:
        scratch = [pltpu.VMEM((C,), jnp.int32),
                   pltpu.VMEM((C,), jnp.int32)] + scratch

    def body(*refs):
        if ew_from_table:
            ewtab, eattr, hwe, src, dst, out = refs[:6]
            ie = refs[6:8]
            rest = refs[8:]
        else:
            ew, hwe, src, dst, out = refs[:5]
            rest = refs[5:]
        isb = rest[0:2]
        idb = rest[2:4]
        eb = rest[4:6]
        b1 = rest[6:8]
        b2 = rest[8:10]
        se = rest[10:12]
        s1 = rest[12:14]
        s2 = rest[14:16]

        cid = lax.axis_index("c")
        sid = lax.axis_index("s")
        wid = cid * _NS + sid

        def issue(k, t):
            base = pl.multiple_of((k * _NW + wid) * C, C)
            pltpu.sync_copy(src.at[pl.ds(base, C)], isb[t])
            pltpu.sync_copy(dst.at[pl.ds(base, C)], idb[t])
            if ew_from_table:
                pltpu.sync_copy(eattr.at[pl.ds(base, C)], ie[t])
                pltpu.make_async_copy(ewtab.at[ie[t]], eb[t], se[t]).start()
            else:
                pltpu.make_async_copy(ew.at[pl.ds(base, C), :],
                                      eb[t], se[t]).start()
            pltpu.make_async_copy(hwe.at[isb[t]], b1[t], s1[t]).start()
            pltpu.make_async_copy(hwe.at[idb[t]], b2[t], s2[t]).start()

        def finish(k, t):
            base = pl.multiple_of((k * _NW + wid) * C, C)
            if ew_from_table:
                pltpu.make_async_copy(ewtab.at[ie[t]], eb[t], se[t]).wait()
            else:
                pltpu.make_async_copy(ew.at[pl.ds(base, C), :],
                                      eb[t], se[t]).wait()
            pltpu.make_async_copy(hwe.at[isb[t]], b1[t], s1[t]).wait()
            pltpu.make_async_copy(hwe.at[idb[t]], b2[t], s2[t]).wait()
            _relu_sum_rows(C, eb[t], b1[t], b2[t])
            pltpu.sync_copy(eb[t], out.at[pl.ds(base, C), :])

        issue(0, 0)

        def pair(j, carry):
            k0 = 2 * j
            issue(k0 + 1, 1)
            finish(k0, 0)
            issue(k0 + 2, 0)
            finish(k0 + 1, 1)
            return carry
        lax.fori_loop(0, (kfull - 1) // 2, pair, 0)
        finish(kfull - 1, 0)

        @pl.when(wid < nextra)
        def _():
            issue(kfull, 1)
            finish(kfull, 1)

    return functools.partial(
        pl.kernel,
        out_type=jax.ShapeDtypeStruct((E, D), jnp.float32),
        mesh=_sc_mesh,
        scratch_types=scratch,
    )(body)


_sca = _make_sca()
_scb = _make_scb(False)


def _mp_block(i, h, e, ew, src, dst, Wm, Wn, We):
    hWm = _node_mm(h, Wm[i])
    if ew is None:
        ew = _edge_mm(e, We[i])
    aggp = _sca(hWm, e, src, dst)
    h_new, hWe = _hnew_fused(h, aggp, Wn[i], We[i])
    e_new = _scb(ew, hWe, src, dst)
    return h_new, e_new


def kernel(x, edge_index, edge_attr, atom_table, bond_table, Wm, Wn, We, pool_p):
    src = edge_index[0].astype(jnp.int32)
    dst = edge_index[1].astype(jnp.int32)
    eattr = edge_attr.astype(jnp.int32)
    atab_pad = jnp.zeros((128, D), jnp.float32).at[:119].set(atom_table)
    h = _emb_mm(x.astype(jnp.int32)[:, None], atab_pad)

    btab_pad = jnp.zeros((128, D), jnp.float32).at[:5].set(bond_table)
    bwe_pad = jnp.zeros((128, D), jnp.float32).at[:5].set(
        _small_mm(bond_table, We[0]))
    e0, ew0 = _e0_mm(eattr[:, None], btab_pad, bwe_pad)

    h, e = _mp_block(0, h, e0, ew0, src, dst, Wm, Wn, We)
    xs = []
    emx = []
    eme = []
    pool_features = []
    for i in range(L):
        xs.append(h)
        p = pool_p[i]
        score = (h @ p) / jnp.linalg.norm(p)
        h = h * jnp.tanh(score)[:, None]
        h, e = _mp_block(1 + i, h, e, None, src, dst, Wm, Wn, We)
        pool_features.append(h)
        gx, ge = h, e
        w = 1.0
        base = 1 + L + i * (i + 1) // 2
        for j in range(i, -1, -1):
            gx = gx + xs[j] / w
            gx, ge = _mp_block(base + j, gx, ge, None, src, dst, Wm, Wn, We)
            xs[j] = xs[j] + gx
            w += 1.0
        emx.append(gx)
        eme.append(ge)
    return (jnp.stack(emx), jnp.stack(eme), jnp.stack(pool_features))


# speedup vs baseline: 4.8056x; 1.0974x over previous
import functools

import jax
import jax.numpy as jnp
from jax import lax
from jax.experimental import pallas as pl
from jax.experimental.pallas import tpu as pltpu
from jax.experimental.pallas import tpu_sc as plsc

N = 10000
E = 160000
D = 128
L = 3

_BN = 2000
_BE = 8000

_NC, _NS = 2, 16
_NW = _NC * _NS
_CA = 64
_CB = 64
_RPT = 624
_REM0, _REMN = _NS * _RPT, N - _NS * _RPT

_sc_mesh = plsc.VectorSubcoreMesh(core_axis_name="c", subcore_axis_name="s")


def _mm_body(x_ref, w_ref, o_ref):
    o_ref[...] = jnp.dot(x_ref[...], w_ref[...],
                         preferred_element_type=jnp.float32)


def _node_mm(x, w):
    return pl.pallas_call(
        _mm_body,
        grid=(N // _BN,),
        in_specs=[
            pl.BlockSpec((_BN, D), lambda i: (i, 0)),
            pl.BlockSpec((D, D), lambda i: (0, 0)),
        ],
        out_specs=pl.BlockSpec((_BN, D), lambda i: (i, 0)),
        out_shape=jax.ShapeDtypeStruct((N, D), jnp.float32),
    )(x, w)


def _edge_mm(x, w):
    return pl.pallas_call(
        _mm_body,
        grid=(E // _BE,),
        in_specs=[
            pl.BlockSpec((_BE, D), lambda i: (i, 0)),
            pl.BlockSpec((D, D), lambda i: (0, 0)),
        ],
        out_specs=pl.BlockSpec((_BE, D), lambda i: (i, 0)),
        out_shape=jax.ShapeDtypeStruct((E, D), jnp.float32),
    )(x, w)


def _small_mm(x, w):
    m, _ = x.shape
    return pl.pallas_call(
        _mm_body,
        in_specs=[pl.BlockSpec(x.shape, lambda: (0, 0)),
                  pl.BlockSpec(w.shape, lambda: (0, 0))],
        out_specs=pl.BlockSpec((m, w.shape[1]), lambda: (0, 0)),
        out_shape=jax.ShapeDtypeStruct((m, w.shape[1]), jnp.float32),
    )(x, w)


def _hnew_body(h_ref, a0_ref, a1_ref, wn_ref, we_ref, hn_ref, hwe_ref):
    hn = jnp.maximum(
        jnp.dot(h_ref[...] + (a0_ref[...] + a1_ref[...]), wn_ref[...],
                preferred_element_type=jnp.float32), 0.0)
    hn_ref[...] = hn
    hwe_ref[...] = jnp.dot(hn, we_ref[...],
                           preferred_element_type=jnp.float32)


def _hnew_fused(h, aggp, wn, we):
    nb = N // _BN
    return pl.pallas_call(
        _hnew_body,
        grid=(nb,),
        in_specs=[
            pl.BlockSpec((_BN, D), lambda i: (i, 0)),
            pl.BlockSpec((_BN, D), lambda i: (i, 0)),
            pl.BlockSpec((_BN, D), lambda i: (i + nb, 0)),
            pl.BlockSpec((D, D), lambda i: (0, 0)),
            pl.BlockSpec((D, D), lambda i: (0, 0)),
        ],
        out_specs=[
            pl.BlockSpec((_BN, D), lambda i: (i, 0)),
            pl.BlockSpec((_BN, D), lambda i: (i, 0)),
        ],
        out_shape=[
            jax.ShapeDtypeStruct((N, D), jnp.float32),
            jax.ShapeDtypeStruct((N, D), jnp.float32),
        ],
    )(h, aggp, aggp, wn, we)


def _emb_body(ids_ref, tab_ref, o_ref):
    oh = (ids_ref[...] == lax.broadcasted_iota(jnp.int32, (1, 128), 1))
    o_ref[...] = jnp.dot(oh.astype(jnp.float32), tab_ref[...],
                         preferred_element_type=jnp.float32)


def _e0_body(ids_ref, bt_ref, bw_ref, e0_ref, ew_ref):
    oh = (ids_ref[...] == lax.broadcasted_iota(jnp.int32, (1, 128), 1))
    ohf = oh.astype(jnp.float32)
    e0_ref[...] = jnp.dot(ohf, bt_ref[...], preferred_element_type=jnp.float32)
    ew_ref[...] = jnp.dot(ohf, bw_ref[...], preferred_element_type=jnp.float32)


def _e0_mm(eattr2d, btab_pad, bwe_pad):
    return pl.pallas_call(
        _e0_body,
        grid=(E // _BE,),
        in_specs=[
            pl.BlockSpec((_BE, 1), lambda i: (i, 0)),
            pl.BlockSpec((128, D), lambda i: (0, 0)),
            pl.BlockSpec((128, D), lambda i: (0, 0)),
        ],
        out_specs=[
            pl.BlockSpec((_BE, D), lambda i: (i, 0)),
            pl.BlockSpec((_BE, D), lambda i: (i, 0)),
        ],
        out_shape=[
            jax.ShapeDtypeStruct((E, D), jnp.float32),
            jax.ShapeDtypeStruct((E, D), jnp.float32),
        ],
    )(eattr2d, btab_pad, bwe_pad)


def _emb_mm(ids2d, tab_pad):
    return pl.pallas_call(
        _emb_body,
        grid=(N // _BN,),
        in_specs=[
            pl.BlockSpec((_BN, 1), lambda i: (i, 0)),
            pl.BlockSpec((128, D), lambda i: (0, 0)),
        ],
        out_specs=pl.BlockSpec((_BN, D), lambda i: (i, 0)),
        out_shape=jax.ShapeDtypeStruct((N, D), jnp.float32),
    )(ids2d, tab_pad)


def _make_sca():
    C = _CA
    kfull = E // C // _NW
    nextra = E // C - kfull * _NW
    assert kfull % 2 == 0 and E % C == 0

    scratch = [
        pltpu.VMEM((2 * C,), jnp.int32),
        pltpu.VMEM((2 * C,), jnp.int32),
        pltpu.VMEM((C,), jnp.int32),
        pltpu.VMEM((C,), jnp.int32),
        pltpu.VMEM((C, D), jnp.float32),
        pltpu.VMEM((C, D), jnp.float32),
        pltpu.VMEM((C, D), jnp.float32),
        pltpu.VMEM((C, D), jnp.float32),
        pltpu.VMEM((C, D), jnp.float32),
        pltpu.VMEM((C, D), jnp.float32),
        pltpu.VMEM_SHARED((N, D), jnp.float32),
        pltpu.SemaphoreType.DMA,
        pltpu.SemaphoreType.DMA,
        pltpu.SemaphoreType.DMA,
        pltpu.SemaphoreType.DMA,
        pltpu.SemaphoreType.DMA,
        pltpu.SemaphoreType.DMA,
    ]

    def body(hwm, esrc, sd, out, *rest):
        isd = rest[0:2]
        sdb = rest[2:4]
        eb = rest[4:6]
        rb = rest[6:8]
        sb = rest[8:10]
        agg = rest[10]
        se = rest[11:13]
        sg = rest[13:15]
        ss = rest[15:17]

        cid = lax.axis_index("c")
        sid = lax.axis_index("s")
        wid = cid * _NS + sid

        def zbody(r, carry):
            for u in range(D // 16):
                rb[0][r, pl.ds(u * 16, 16)] = jnp.zeros((16,), jnp.float32)
            return carry
        lax.fori_loop(0, C, zbody, 0)
        nz, rem = _RPT // C, _RPT % C
        for j in range(nz):
            pltpu.sync_copy(rb[0].at[pl.ds(0, C)],
                            agg.at[pl.ds(sid * _RPT + j * C, C)])
        if rem:
            pltpu.sync_copy(rb[0].at[pl.ds(0, rem)],
                            agg.at[pl.ds(sid * _RPT + nz * C, rem)])

        @pl.when(sid == _NS - 1)
        def _():
            pltpu.sync_copy(rb[0].at[pl.ds(0, _REMN)],
                            agg.at[pl.ds(_REM0, _REMN)])
        plsc.subcore_barrier()

        def issue(k, t):
            g = k * _NW + wid
            base = pl.multiple_of(g * C, C)
            pltpu.sync_copy(sd.at[pl.ds(2 * base, 2 * C)], isd[t])
            pltpu.make_async_copy(esrc.at[pl.ds(base, C), :],
                                  eb[t], se[t]).start()
            pltpu.make_async_copy(hwm.at[isd[t].at[pl.ds(0, C)]],
                                  rb[t], sg[t]).start()

        def finish(t, guard_scatter=True):
            pltpu.make_async_copy(esrc.at[pl.ds(0, C), :],
                                  eb[t], se[t]).wait()
            pltpu.make_async_copy(hwm.at[isd[t].at[pl.ds(0, C)]],
                                  rb[t], sg[t]).wait()
            if guard_scatter:
                pltpu.make_async_copy(sb[t], agg.at[sdb[t]], ss[t]).wait()
            for u in range(C // 16):
                sdb[t][pl.ds(u * 16, 16)] = isd[t][pl.ds(C + u * 16, 16)]

            def cbody(r, carry):
                for u in range(D // 16):
                    sl = pl.ds(u * 16, 16)
                    sb[t][r, sl] = jnp.maximum(eb[t][r, sl] + rb[t][r, sl],
                                               0.0)
                return carry
            lax.fori_loop(0, C, cbody, 0)
            pltpu.async_copy(sb[t], agg.at[sdb[t]], ss[t], add=True)

        issue(0, 0)
        issue(1, 1)
        finish(0, guard_scatter=False)
        issue(2, 0)
        finish(1, guard_scatter=False)

        def pair(j, carry):
            k0 = 2 * j
            issue(k0 + 3, 1)
            finish(0)
            issue(k0 + 4, 0)
            finish(1)
            return carry
        lax.fori_loop(0, kfull // 2 - 2, pair, 0)
        issue(kfull - 1, 1)
        finish(0)
        if nextra:
            @pl.when(wid < nextra)
            def _():
                issue(kfull, 0)
            finish(1)

            @pl.when(wid < nextra)
            def _():
                finish(0)
        else:
            finish(1)

        pltpu.make_async_copy(sb[0], agg.at[sdb[0]], ss[0]).wait()
        pltpu.make_async_copy(sb[1], agg.at[sdb[1]], ss[1]).wait()

        plsc.subcore_barrier()
        for off, sz in [(0, 128), (128, 128), (256, 128), (384, 128),
                        (512, 112)]:
            r0 = sid * _RPT + off
            pltpu.sync_copy(agg.at[pl.ds(r0, sz)],
                            out.at[pl.ds(cid * N + r0, sz)])

        @pl.when(sid == _NS - 1)
        def _():
            pltpu.sync_copy(agg.at[pl.ds(_REM0, _REMN)],
                            out.at[pl.ds(cid * N + _REM0, _REMN)])

    return functools.partial(
        pl.kernel,
        out_type=jax.ShapeDtypeStruct((2 * N, D), jnp.float32),
        mesh=_sc_mesh,
        scratch_types=scratch,
    )(body)


def _make_scb():
    C = _CB
    kfull = E // C // _NW
    nextra = E // C - kfull * _NW
    assert kfull % 2 == 0 and E % C == 0
    scratch = [
        pltpu.VMEM((2 * C,), jnp.int32),
        pltpu.VMEM((2 * C,), jnp.int32),
        pltpu.VMEM((C, D), jnp.float32),
        pltpu.VMEM((C, D), jnp.float32),
        pltpu.VMEM((C, D), jnp.float32),
        pltpu.VMEM((C, D), jnp.float32),
        pltpu.VMEM((C, D), jnp.float32),
        pltpu.VMEM((C, D), jnp.float32),
        pltpu.VMEM((C, D), jnp.float32),
        pltpu.VMEM((C, D), jnp.float32),
        pltpu.SemaphoreType.DMA,
        pltpu.SemaphoreType.DMA,
        pltpu.SemaphoreType.DMA,
        pltpu.SemaphoreType.DMA,
        pltpu.SemaphoreType.DMA,
        pltpu.SemaphoreType.DMA,
        pltpu.SemaphoreType.DMA,
        pltpu.SemaphoreType.DMA,
    ]

    def body(ew, hwe, sd, out, *rest):
        isd = rest[0:2]
        eb = rest[2:4]
        b1 = rest[4:6]
        b2 = rest[6:8]
        ob = rest[8:10]
        se = rest[10:12]
        s1 = rest[12:14]
        s2 = rest[14:16]
        so = rest[16:18]

        cid = lax.axis_index("c")
        sid = lax.axis_index("s")
        wid = cid * _NS + sid

        def issue(k, t):
            base = pl.multiple_of((k * _NW + wid) * C, C)
            pltpu.sync_copy(sd.at[pl.ds(2 * base, 2 * C)], isd[t])
            pltpu.make_async_copy(ew.at[pl.ds(base, C), :],
                                  eb[t], se[t]).start()
            pltpu.make_async_copy(hwe.at[isd[t].at[pl.ds(0, C)]],
                                  b1[t], s1[t]).start()
            pltpu.make_async_copy(hwe.at[isd[t].at[pl.ds(C, C)]],
                                  b2[t], s2[t]).start()

        def finish(k, t, guard_store=True):
            base = pl.multiple_of((k * _NW + wid) * C, C)
            pltpu.make_async_copy(ew.at[pl.ds(0, C), :],
                                  eb[t], se[t]).wait()
            pltpu.make_async_copy(hwe.at[isd[t].at[pl.ds(0, C)]],
                                  b1[t], s1[t]).wait()
            pltpu.make_async_copy(hwe.at[isd[t].at[pl.ds(C, C)]],
                                  b2[t], s2[t]).wait()
            if guard_store:
                pltpu.make_async_copy(ob[t], out.at[pl.ds(0, C), :],
                                      so[t]).wait()

            def cbody(r, carry):
                for u in range(D // 16):
                    sl = pl.ds(u * 16, 16)
                    ob[t][r, sl] = jnp.maximum(
                        eb[t][r, sl] + (b1[t][r, sl] + b2[t][r, sl]), 0.0)
                return carry
            lax.fori_loop(0, C, cbody, 0)
            pltpu.make_async_copy(ob[t], out.at[pl.ds(base, C), :],
                                  so[t]).start()

        issue(0, 0)
        issue(1, 1)
        finish(0, 0, guard_store=False)
        issue(2, 0)
        finish(1, 1, guard_store=False)

        def pair(j, carry):
            k0 = 2 * j
            issue(k0 + 3, 1)
            finish(k0 + 2, 0)
            issue(k0 + 4, 0)
            finish(k0 + 3, 1)
            return carry
        lax.fori_loop(0, kfull // 2 - 2, pair, 0)
        issue(kfull - 1, 1)
        finish(kfull - 2, 0)
        if nextra:
            @pl.when(wid < nextra)
            def _():
                issue(kfull, 0)
            finish(kfull - 1, 1)

            @pl.when(wid < nextra)
            def _():
                finish(kfull, 0)
        else:
            finish(kfull - 1, 1)

        pltpu.make_async_copy(ob[0], out.at[pl.ds(0, C), :], so[0]).wait()
        pltpu.make_async_copy(ob[1], out.at[pl.ds(0, C), :], so[1]).wait()

    return functools.partial(
        pl.kernel,
        out_type=jax.ShapeDtypeStruct((E, D), jnp.float32),
        mesh=_sc_mesh,
        scratch_types=scratch,
    )(body)


_sca = _make_sca()
_scb = _make_scb()


def _mp_block(i, h, e, ew, sd, Wm, Wn, We):
    hWm = _node_mm(h, Wm[i])
    if ew is None:
        ew = _edge_mm(e, We[i])
    aggp = _sca(hWm, e, sd)
    h_new, hWe = _hnew_fused(h, aggp, Wn[i], We[i])
    e_new = _scb(ew, hWe, sd)
    return h_new, e_new


def kernel(x, edge_index, edge_attr, atom_table, bond_table, Wm, Wn, We, pool_p):
    src = edge_index[0].astype(jnp.int32)
    dst = edge_index[1].astype(jnp.int32)
    sd = jnp.stack([src.reshape(E // _CA, _CA),
                    dst.reshape(E // _CA, _CA)], axis=1).reshape(2 * E)
    eattr = edge_attr.astype(jnp.int32)
    atab_pad = jnp.zeros((128, D), jnp.float32).at[:119].set(atom_table)
    h = _emb_mm(x.astype(jnp.int32)[:, None], atab_pad)

    btab_pad = jnp.zeros((128, D), jnp.float32).at[:5].set(bond_table)
    bwe_pad = jnp.zeros((128, D), jnp.float32).at[:5].set(
        _small_mm(bond_table, We[0]))
    e0, ew0 = _e0_mm(eattr[:, None], btab_pad, bwe_pad)

    h, e = _mp_block(0, h, e0, ew0, sd, Wm, Wn, We)
    xs = []
    emx = []
    eme = []
    pool_features = []
    for i in range(L):
        xs.append(h)
        p = pool_p[i]
        score = (h @ p) / jnp.linalg.norm(p)
        h = h * jnp.tanh(score)[:, None]
        h, e = _mp_block(1 + i, h, e, None, sd, Wm, Wn, We)
        pool_features.append(h)
        gx, ge = h, e
        w = 1.0
        base = 1 + L + i * (i + 1) // 2
        for j in range(i, -1, -1):
            gx = gx + xs[j] / w
            gx, ge = _mp_block(base + j, gx, ge, None, sd, Wm, Wn, We)
            xs[j] = xs[j] + gx
            w += 1.0
        emx.append(gx)
        eme.append(ge)
    return (jnp.stack(emx), jnp.stack(eme), jnp.stack(pool_features))


# speedup vs baseline: 5.1185x; 1.0651x over previous
import functools

import jax
import jax.numpy as jnp
from jax import lax
from jax.experimental import pallas as pl
from jax.experimental.pallas import tpu as pltpu
from jax.experimental.pallas import tpu_sc as plsc

N = 10000
E = 160000
D = 128
L = 3

_BN = 2000
_BE = 8000

_NC, _NS = 2, 16
_NW = _NC * _NS
_CA = 64
_CB = 64
_RPT = 624
_REM0, _REMN = _NS * _RPT, N - _NS * _RPT

_sc_mesh = plsc.VectorSubcoreMesh(core_axis_name="c", subcore_axis_name="s")


def _mm_body(x_ref, w_ref, o_ref):
    o_ref[...] = jnp.dot(x_ref[...], w_ref[...],
                         preferred_element_type=jnp.float32)


def _node_mm(x, w):
    return pl.pallas_call(
        _mm_body,
        grid=(N // _BN,),
        in_specs=[
            pl.BlockSpec((_BN, D), lambda i: (i, 0)),
            pl.BlockSpec((D, D), lambda i: (0, 0)),
        ],
        out_specs=pl.BlockSpec((_BN, D), lambda i: (i, 0)),
        out_shape=jax.ShapeDtypeStruct((N, D), jnp.float32),
    )(x, w)


def _edge_mm(x, w):
    return pl.pallas_call(
        _mm_body,
        grid=(E // _BE,),
        in_specs=[
            pl.BlockSpec((_BE, D), lambda i: (i, 0)),
            pl.BlockSpec((D, D), lambda i: (0, 0)),
        ],
        out_specs=pl.BlockSpec((_BE, D), lambda i: (i, 0)),
        out_shape=jax.ShapeDtypeStruct((E, D), jnp.float32),
    )(x, w)


def _small_mm(x, w):
    m, _ = x.shape
    return pl.pallas_call(
        _mm_body,
        in_specs=[pl.BlockSpec(x.shape, lambda: (0, 0)),
                  pl.BlockSpec(w.shape, lambda: (0, 0))],
        out_specs=pl.BlockSpec((m, w.shape[1]), lambda: (0, 0)),
        out_shape=jax.ShapeDtypeStruct((m, w.shape[1]), jnp.float32),
    )(x, w)


def _hnew_body(h_ref, a0_ref, a1_ref, wn_ref, we_ref, hn_ref, hwe_ref):
    hn = jnp.maximum(
        jnp.dot(h_ref[...] + (a0_ref[...] + a1_ref[...]), wn_ref[...],
                preferred_element_type=jnp.float32), 0.0)
    hn_ref[...] = hn
    hwe_ref[...] = jnp.dot(hn, we_ref[...],
                           preferred_element_type=jnp.float32)


def _hnew_fused(h, aggp, wn, we):
    nb = N // _BN
    return pl.pallas_call(
        _hnew_body,
        grid=(nb,),
        in_specs=[
            pl.BlockSpec((_BN, D), lambda i: (i, 0)),
            pl.BlockSpec((_BN, D), lambda i: (i, 0)),
            pl.BlockSpec((_BN, D), lambda i: (i + nb, 0)),
            pl.BlockSpec((D, D), lambda i: (0, 0)),
            pl.BlockSpec((D, D), lambda i: (0, 0)),
        ],
        out_specs=[
            pl.BlockSpec((_BN, D), lambda i: (i, 0)),
            pl.BlockSpec((_BN, D), lambda i: (i, 0)),
        ],
        out_shape=[
            jax.ShapeDtypeStruct((N, D), jnp.float32),
            jax.ShapeDtypeStruct((N, D), jnp.float32),
        ],
    )(h, aggp, aggp, wn, we)


def _emb_body(ids_ref, tab_ref, o_ref):
    oh = (ids_ref[...] == lax.broadcasted_iota(jnp.int32, (1, 128), 1))
    o_ref[...] = jnp.dot(oh.astype(jnp.float32), tab_ref[...],
                         preferred_element_type=jnp.float32)


def _e0_body(ids_ref, bt_ref, bw_ref, e0_ref, ew_ref):
    oh = (ids_ref[...] == lax.broadcasted_iota(jnp.int32, (1, 128), 1))
    ohf = oh.astype(jnp.float32)
    e0_ref[...] = jnp.dot(ohf, bt_ref[...], preferred_element_type=jnp.float32)
    ew_ref[...] = jnp.dot(ohf, bw_ref[...], preferred_element_type=jnp.float32)


def _e0_mm(eattr2d, btab_pad, bwe_pad):
    return pl.pallas_call(
        _e0_body,
        grid=(E // _BE,),
        in_specs=[
            pl.BlockSpec((_BE, 1), lambda i: (i, 0)),
            pl.BlockSpec((128, D), lambda i: (0, 0)),
            pl.BlockSpec((128, D), lambda i: (0, 0)),
        ],
        out_specs=[
            pl.BlockSpec((_BE, D), lambda i: (i, 0)),
            pl.BlockSpec((_BE, D), lambda i: (i, 0)),
        ],
        out_shape=[
            jax.ShapeDtypeStruct((E, D), jnp.float32),
            jax.ShapeDtypeStruct((E, D), jnp.float32),
        ],
    )(eattr2d, btab_pad, bwe_pad)


def _emb_mm(ids2d, tab_pad):
    return pl.pallas_call(
        _emb_body,
        grid=(N // _BN,),
        in_specs=[
            pl.BlockSpec((_BN, 1), lambda i: (i, 0)),
            pl.BlockSpec((128, D), lambda i: (0, 0)),
        ],
        out_specs=pl.BlockSpec((_BN, D), lambda i: (i, 0)),
        out_shape=jax.ShapeDtypeStruct((N, D), jnp.float32),
    )(ids2d, tab_pad)


def _make_sca():
    C = _CA
    kfull = E // C // _NW
    nextra = E // C - kfull * _NW
    assert kfull % 2 == 0 and E % C == 0

    scratch = [
        pltpu.VMEM((2 * C,), jnp.int32),
        pltpu.VMEM((2 * C,), jnp.int32),
        pltpu.VMEM((C,), jnp.int32),
        pltpu.VMEM((C,), jnp.int32),
        pltpu.VMEM((C, D), jnp.float32),
        pltpu.VMEM((C, D), jnp.float32),
        pltpu.VMEM((C, D), jnp.float32),
        pltpu.VMEM((C, D), jnp.float32),
        pltpu.VMEM((C, D), jnp.float32),
        pltpu.VMEM((C, D), jnp.float32),
        pltpu.VMEM_SHARED((N, D), jnp.float32),
        pltpu.SemaphoreType.DMA,
        pltpu.SemaphoreType.DMA,
        pltpu.SemaphoreType.DMA,
        pltpu.SemaphoreType.DMA,
        pltpu.SemaphoreType.DMA,
        pltpu.SemaphoreType.DMA,
        pltpu.SemaphoreType.DMA,
        pltpu.SemaphoreType.DMA,
    ]

    def body(hwm, esrc, sd, out, *rest):
        isd = rest[0:2]
        sdb = rest[2:4]
        eb = rest[4:6]
        rb = rest[6:8]
        sb = rest[8:10]
        agg = rest[10]
        se = rest[11:13]
        sg = rest[13:15]
        ss = rest[15:17]
        si = rest[17:19]

        cid = lax.axis_index("c")
        sid = lax.axis_index("s")
        wid = cid * _NS + sid

        def zbody(r, carry):
            for u in range(D // 16):
                rb[0][r, pl.ds(u * 16, 16)] = jnp.zeros((16,), jnp.float32)
            return carry
        lax.fori_loop(0, C, zbody, 0)
        nz, rem = _RPT // C, _RPT % C
        for j in range(nz):
            pltpu.sync_copy(rb[0].at[pl.ds(0, C)],
                            agg.at[pl.ds(sid * _RPT + j * C, C)])
        if rem:
            pltpu.sync_copy(rb[0].at[pl.ds(0, rem)],
                            agg.at[pl.ds(sid * _RPT + nz * C, rem)])

        @pl.when(sid == _NS - 1)
        def _():
            pltpu.sync_copy(rb[0].at[pl.ds(0, _REMN)],
                            agg.at[pl.ds(_REM0, _REMN)])
        plsc.subcore_barrier()

        def issue(k, t, idx_sync=False):
            base = pl.multiple_of((k * _NW + wid) * C, C)
            if idx_sync:
                pltpu.sync_copy(sd.at[pl.ds(2 * base, 2 * C)], isd[t])
            else:
                pltpu.make_async_copy(sd.at[pl.ds(0, 2 * C)],
                                      isd[t], si[t]).wait()
            pltpu.make_async_copy(esrc.at[pl.ds(base, C), :],
                                  eb[t], se[t]).start()
            pltpu.make_async_copy(hwm.at[isd[t].at[pl.ds(0, C)]],
                                  rb[t], sg[t]).start()

        def finish(t, guard_scatter=True, prefetch=None, prefetch_cond=None):
            pltpu.make_async_copy(esrc.at[pl.ds(0, C), :],
                                  eb[t], se[t]).wait()
            pltpu.make_async_copy(hwm.at[isd[t].at[pl.ds(0, C)]],
                                  rb[t], sg[t]).wait()
            if guard_scatter:
                pltpu.make_async_copy(sb[t], agg.at[sdb[t]], ss[t]).wait()
            for u in range(C // 16):
                sdb[t][pl.ds(u * 16, 16)] = isd[t][pl.ds(C + u * 16, 16)]
            if prefetch is not None:
                def start_pf():
                    base2 = pl.multiple_of((prefetch * _NW + wid) * C, C)
                    pltpu.make_async_copy(sd.at[pl.ds(2 * base2, 2 * C)],
                                          isd[t], si[t]).start()
                if prefetch_cond is None:
                    start_pf()
                else:
                    pl.when(prefetch_cond)(start_pf)

            def cbody(r, carry):
                for u in range(D // 16):
                    sl = pl.ds(u * 16, 16)
                    sb[t][r, sl] = jnp.maximum(eb[t][r, sl] + rb[t][r, sl],
                                               0.0)
                return carry
            lax.fori_loop(0, C, cbody, 0)
            pltpu.async_copy(sb[t], agg.at[sdb[t]], ss[t], add=True)

        issue(0, 0, idx_sync=True)
        issue(1, 1, idx_sync=True)
        finish(0, guard_scatter=False, prefetch=2)
        issue(2, 0)
        finish(1, guard_scatter=False, prefetch=3)

        def pair(j, carry):
            k0 = 2 * j
            issue(k0 + 3, 1)
            finish(0, prefetch=k0 + 4)
            issue(k0 + 4, 0)
            finish(1, prefetch=k0 + 5)
            return carry
        lax.fori_loop(0, kfull // 2 - 2, pair, 0)
        issue(kfull - 1, 1)
        if nextra:
            finish(0, prefetch=kfull, prefetch_cond=wid < nextra)

            @pl.when(wid < nextra)
            def _():
                issue(kfull, 0)
            finish(1)

            @pl.when(wid < nextra)
            def _():
                finish(0)
        else:
            finish(0)
            finish(1)

        pltpu.make_async_copy(sb[0], agg.at[sdb[0]], ss[0]).wait()
        pltpu.make_async_copy(sb[1], agg.at[sdb[1]], ss[1]).wait()

        plsc.subcore_barrier()
        for off, sz in [(0, 128), (128, 128), (256, 128), (384, 128),
                        (512, 112)]:
            r0 = sid * _RPT + off
            pltpu.sync_copy(agg.at[pl.ds(r0, sz)],
                            out.at[pl.ds(cid * N + r0, sz)])

        @pl.when(sid == _NS - 1)
        def _():
            pltpu.sync_copy(agg.at[pl.ds(_REM0, _REMN)],
                            out.at[pl.ds(cid * N + _REM0, _REMN)])

    return functools.partial(
        pl.kernel,
        out_type=jax.ShapeDtypeStruct((2 * N, D), jnp.float32),
        mesh=_sc_mesh,
        scratch_types=scratch,
    )(body)


def _make_scb():
    C = _CB
    kfull = E // C // _NW
    nextra = E // C - kfull * _NW
    assert kfull % 2 == 0 and E % C == 0
    scratch = [
        pltpu.VMEM((2 * C,), jnp.int32),
        pltpu.VMEM((2 * C,), jnp.int32),
        pltpu.VMEM((C, D), jnp.float32),
        pltpu.VMEM((C, D), jnp.float32),
        pltpu.VMEM((C, D), jnp.float32),
        pltpu.VMEM((C, D), jnp.float32),
        pltpu.VMEM((C, D), jnp.float32),
        pltpu.VMEM((C, D), jnp.float32),
        pltpu.VMEM((C, D), jnp.float32),
        pltpu.VMEM((C, D), jnp.float32),
        pltpu.SemaphoreType.DMA,
        pltpu.SemaphoreType.DMA,
        pltpu.SemaphoreType.DMA,
        pltpu.SemaphoreType.DMA,
        pltpu.SemaphoreType.DMA,
        pltpu.SemaphoreType.DMA,
        pltpu.SemaphoreType.DMA,
        pltpu.SemaphoreType.DMA,
        pltpu.SemaphoreType.DMA,
        pltpu.SemaphoreType.DMA,
    ]

    def body(ew, hwe, sd, out, *rest):
        isd = rest[0:2]
        eb = rest[2:4]
        b1 = rest[4:6]
        b2 = rest[6:8]
        ob = rest[8:10]
        se = rest[10:12]
        s1 = rest[12:14]
        s2 = rest[14:16]
        so = rest[16:18]
        si = rest[18:20]

        cid = lax.axis_index("c")
        sid = lax.axis_index("s")
        wid = cid * _NS + sid

        def issue(k, t, idx_sync=False):
            base = pl.multiple_of((k * _NW + wid) * C, C)
            if idx_sync:
                pltpu.sync_copy(sd.at[pl.ds(2 * base, 2 * C)], isd[t])
            else:
                pltpu.make_async_copy(sd.at[pl.ds(0, 2 * C)],
                                      isd[t], si[t]).wait()
            pltpu.make_async_copy(ew.at[pl.ds(base, C), :],
                                  eb[t], se[t]).start()
            pltpu.make_async_copy(hwe.at[isd[t].at[pl.ds(0, C)]],
                                  b1[t], s1[t]).start()
            pltpu.make_async_copy(hwe.at[isd[t].at[pl.ds(C, C)]],
                                  b2[t], s2[t]).start()

        def finish(k, t, guard_store=True, prefetch=None, prefetch_cond=None):
            base = pl.multiple_of((k * _NW + wid) * C, C)
            pltpu.make_async_copy(ew.at[pl.ds(0, C), :],
                                  eb[t], se[t]).wait()
            pltpu.make_async_copy(hwe.at[isd[t].at[pl.ds(0, C)]],
                                  b1[t], s1[t]).wait()
            pltpu.make_async_copy(hwe.at[isd[t].at[pl.ds(C, C)]],
                                  b2[t], s2[t]).wait()
            if guard_store:
                pltpu.make_async_copy(ob[t], out.at[pl.ds(0, C), :],
                                      so[t]).wait()
            if prefetch is not None:
                def start_pf():
                    base2 = pl.multiple_of((prefetch * _NW + wid) * C, C)
                    pltpu.make_async_copy(sd.at[pl.ds(2 * base2, 2 * C)],
                                          isd[t], si[t]).start()
                if prefetch_cond is None:
                    start_pf()
                else:
                    pl.when(prefetch_cond)(start_pf)

            def cbody(r, carry):
                for u in range(D // 16):
                    sl = pl.ds(u * 16, 16)
                    ob[t][r, sl] = jnp.maximum(
                        eb[t][r, sl] + (b1[t][r, sl] + b2[t][r, sl]), 0.0)
                return carry
            lax.fori_loop(0, C, cbody, 0)
            pltpu.make_async_copy(ob[t], out.at[pl.ds(base, C), :],
                                  so[t]).start()

        issue(0, 0, idx_sync=True)
        issue(1, 1, idx_sync=True)
        finish(0, 0, guard_store=False, prefetch=2)
        issue(2, 0)
        finish(1, 1, guard_store=False, prefetch=3)

        def pair(j, carry):
            k0 = 2 * j
            issue(k0 + 3, 1)
            finish(k0 + 2, 0, prefetch=k0 + 4)
            issue(k0 + 4, 0)
            finish(k0 + 3, 1, prefetch=k0 + 5)
            return carry
        lax.fori_loop(0, kfull // 2 - 2, pair, 0)
        issue(kfull - 1, 1)
        if nextra:
            finish(kfull - 2, 0, prefetch=kfull, prefetch_cond=wid < nextra)

            @pl.when(wid < nextra)
            def _():
                issue(kfull, 0)
            finish(kfull - 1, 1)

            @pl.when(wid < nextra)
            def _():
                finish(kfull, 0)
        else:
            finish(kfull - 2, 0)
            finish(kfull - 1, 1)

        pltpu.make_async_copy(ob[0], out.at[pl.ds(0, C), :], so[0]).wait()
        pltpu.make_async_copy(ob[1], out.at[pl.ds(0, C), :], so[1]).wait()

    return functools.partial(
        pl.kernel,
        out_type=jax.ShapeDtypeStruct((E, D), jnp.float32),
        mesh=_sc_mesh,
        scratch_types=scratch,
    )(body)


_sca = _make_sca()
_scb = _make_scb()


def _mp_block(i, h, e, ew, sd, Wm, Wn, We):
    hWm = _node_mm(h, Wm[i])
    if ew is None:
        ew = _edge_mm(e, We[i])
    aggp = _sca(hWm, e, sd)
    h_new, hWe = _hnew_fused(h, aggp, Wn[i], We[i])
    e_new = _scb(ew, hWe, sd)
    return h_new, e_new


def kernel(x, edge_index, edge_attr, atom_table, bond_table, Wm, Wn, We, pool_p):
    src = edge_index[0].astype(jnp.int32)
    dst = edge_index[1].astype(jnp.int32)
    sd = jnp.stack([src.reshape(E // _CA, _CA),
                    dst.reshape(E // _CA, _CA)], axis=1).reshape(2 * E)
    eattr = edge_attr.astype(jnp.int32)
    atab_pad = jnp.zeros((128, D), jnp.float32).at[:119].set(atom_table)
    h = _emb_mm(x.astype(jnp.int32)[:, None], atab_pad)

    btab_pad = jnp.zeros((128, D), jnp.float32).at[:5].set(bond_table)
    bwe_pad = jnp.zeros((128, D), jnp.float32).at[:5].set(
        _small_mm(bond_table, We[0]))
    e0, ew0 = _e0_mm(eattr[:, None], btab_pad, bwe_pad)

    h, e = _mp_block(0, h, e0, ew0, sd, Wm, Wn, We)
    xs = []
    emx = []
    eme = []
    pool_features = []
    for i in range(L):
        xs.append(h)
        p = pool_p[i]
        score = (h @ p) / jnp.linalg.norm(p)
        h = h * jnp.tanh(score)[:, None]
        h, e = _mp_block(1 + i, h, e, None, sd, Wm, Wn, We)
        pool_features.append(h)
        gx, ge = h, e
        w = 1.0
        base = 1 + L + i * (i + 1) // 2
        for j in range(i, -1, -1):
            gx = gx + xs[j] / w
            gx, ge = _mp_block(base + j, gx, ge, None, sd, Wm, Wn, We)
            xs[j] = xs[j] + gx
            w += 1.0
        emx.append(gx)
        eme.append(ge)
    return (jnp.stack(emx), jnp.stack(eme), jnp.stack(pool_features))
